# baseline (speedup 1.0000x reference)
import functools

import jax
import jax.numpy as jnp
import numpy as np
from jax import lax
from jax.experimental import pallas as pl
from jax.experimental.pallas import tpu as pltpu
from jax.experimental.pallas import tpu_sc as plsc

_B, _L, _C = 16, 4096, 128


def _spiral_pattern(L):
    PI = float(np.arccos(0.0) * 2.0)
    size = np.sqrt(L / (PI / 4.0 * 0.7))
    size = np.round(size / 2.0)
    size = int(size * 2 + 1)
    rnge = (np.arange(size, dtype=np.float32) - np.float32(size / 2.0)
            + np.float32(0.5)).astype(np.float32)
    x1, x2 = np.meshgrid(rnge, rnge)
    r = np.sqrt(np.abs(x1 * x1 + x2 * x2), dtype=np.float32)
    with np.errstate(invalid="ignore", divide="ignore"):
        phi = np.arccos((x1 / r).astype(np.float32)).astype(np.float32)
    phi = np.where(np.isnan(phi), np.float32(0.0), phi)
    phi = (phi * np.sign(x2)).astype(np.float32)
    is_pi = (np.logical_and(x2 == 0, x1 < 0).astype(np.float32)
             * np.float32(PI)).astype(np.float32)
    phi = (phi + is_pi).astype(np.float32)
    phi2 = (np.round(r).astype(np.float32) * np.float32(2.0)
            * np.float32(PI) + phi).astype(np.float32)
    idx = np.argsort(phi2.reshape(-1), kind="stable")[:L]
    return size, idx.astype(np.int64)


_SIZE, _IDX = _spiral_pattern(_L)
_S2 = _SIZE * _SIZE

_NW = 32
_CHUNK = 128

_rows = (np.arange(_B, dtype=np.int64)[:, None] * _S2 + _IDX[None, :]).reshape(-1)
_N_SCHUNK = (_B * _L) // (_NW * _CHUNK)
_SIDX_NP = _rows.reshape(_NW, _N_SCHUNK, _CHUNK).astype(np.int32)

_mask = np.ones(_S2, dtype=bool)
_mask[_IDX] = False
_comp = np.nonzero(_mask)[0].astype(np.int64)
_HEAD = int(_IDX.min())
_TAIL0 = int(_IDX.max()) + 1
_TAILN = _S2 - _TAIL0
assert _HEAD == 646 and _TAILN == 734
_int_comp = _comp[(_comp >= _HEAD) & (_comp < _TAIL0)]
_zrows = (np.arange(_B, dtype=np.int64)[:, None]) * _S2 + _int_comp[None, :]
_zrows = _zrows.reshape(-1)
_N_ZCHUNK = -(-len(_zrows) // (_NW * _CHUNK))
_pad = _N_ZCHUNK * _NW * _CHUNK - len(_zrows)
_zrows = np.concatenate([_zrows, _zrows[:_pad]])
_ZIDX_NP = _zrows.reshape(_NW, _N_ZCHUNK, _CHUNK).astype(np.int32)

_ROWS_PER_W = _N_SCHUNK * _CHUNK
_NBUF = 6
_GROUPS = [list(range(0, 6)), list(range(6, 12)), list(range(12, 16))]


def _make_scatter():
    mesh = plsc.VectorSubcoreMesh(core_axis_name="c", subcore_axis_name="s")

    @functools.partial(
        pl.kernel,
        mesh=mesh,
        compiler_params=pltpu.CompilerParams(use_tc_tiling_on_sc=False),
        out_type=jax.ShapeDtypeStruct((_B * _S2, _C), jnp.float32),
        scratch_types=[
            pltpu.VMEM((_N_SCHUNK, _CHUNK), jnp.int32),
            pltpu.VMEM((_N_ZCHUNK, _CHUNK), jnp.int32),
            pltpu.VMEM((_CHUNK, _C), jnp.float32),
        ] + [pltpu.VMEM((_CHUNK, _C), jnp.float32)] * _NBUF + [
            pltpu.SemaphoreType.DMA,
            pltpu.SemaphoreType.DMA,
            pltpu.SemaphoreType.DMA,
        ],
    )
    def scatter(in_hbm, sidx_hbm, zidx_hbm, zeros_hbm, out_hbm,
                sidx_v, zidx_v, zbuf_v, *rest):
        bufs = list(rest[:_NBUF])
        sem_in, sem_out, sem_z = rest[_NBUF:_NBUF + 3]

        nc = 2
        wid = lax.axis_index("s") * nc + lax.axis_index("c")
        base = wid * _ROWS_PER_W

        m0 = pltpu.async_copy(sidx_hbm.at[wid], sidx_v, sem_z)
        m1 = pltpu.async_copy(zidx_hbm.at[wid], zidx_v, sem_z)
        m2 = pltpu.async_copy(zeros_hbm, zbuf_v, sem_z)
        m0.wait(); m1.wait(); m2.wait()

        bz = (wid // 2) * _S2 + (wid % 2) * _TAIL0
        zd = [pltpu.async_copy(
                  zbuf_v, out_hbm.at[pl.ds(bz + i * _CHUNK, _CHUNK)], sem_z)
              for i in range(5)]
        zd.append(pltpu.async_copy(
            zbuf_v.at[pl.ds(0, 6)], out_hbm.at[pl.ds(bz + 640, 6)], sem_z))

        @pl.when(wid % 2 == 1)
        def _tail_extra():
            pltpu.async_copy(
                zbuf_v.at[pl.ds(0, 88)], out_hbm.at[pl.ds(bz + 646, 88)],
                sem_z).wait()

        for gi, grp in enumerate(_GROUPS):
            st = [pltpu.async_copy(
                      in_hbm.at[pl.ds(base + c * _CHUNK, _CHUNK)],
                      bufs[j], sem_in)
                  for j, c in enumerate(grp)]
            if gi == 0:
                for d in zd:
                    d.wait()
            for d in st:
                d.wait()
            plsc.subcore_barrier()
            sc = [pltpu.async_copy(
                      bufs[j], out_hbm.at[sidx_v.at[c]], sem_out)
                  for j, c in enumerate(grp)]
            if gi == len(_GROUPS) - 1:
                sc += [pltpu.async_copy(
                           zbuf_v, out_hbm.at[zidx_v.at[z]], sem_out)
                       for z in range(_N_ZCHUNK)]
            for d in sc:
                d.wait()
            plsc.subcore_barrier()

    return scatter


_scatter = _make_scatter()


def kernel(inputs):
    B, L, C = inputs.shape
    flat = inputs.reshape(B * L, C)
    sidx = jnp.asarray(_SIDX_NP)
    zidx = jnp.asarray(_ZIDX_NP)
    zeros = jnp.zeros((_CHUNK, _C), dtype=jnp.float32)
    out = _scatter(flat, sidx, zidx, zeros)
    return out.reshape(B, _SIZE, _SIZE, C)


# speedup vs baseline: 1.0219x; 1.0219x over previous
import functools

import jax
import jax.numpy as jnp
import numpy as np
from jax import lax
from jax.experimental import pallas as pl
from jax.experimental.pallas import tpu as pltpu
from jax.experimental.pallas import tpu_sc as plsc

_B, _L, _C = 16, 4096, 128


def _spiral_pattern(L):
    PI = float(np.arccos(0.0) * 2.0)
    size = np.sqrt(L / (PI / 4.0 * 0.7))
    size = np.round(size / 2.0)
    size = int(size * 2 + 1)
    rnge = (np.arange(size, dtype=np.float32) - np.float32(size / 2.0)
            + np.float32(0.5)).astype(np.float32)
    x1, x2 = np.meshgrid(rnge, rnge)
    r = np.sqrt(np.abs(x1 * x1 + x2 * x2), dtype=np.float32)
    with np.errstate(invalid="ignore", divide="ignore"):
        phi = np.arccos((x1 / r).astype(np.float32)).astype(np.float32)
    phi = np.where(np.isnan(phi), np.float32(0.0), phi)
    phi = (phi * np.sign(x2)).astype(np.float32)
    is_pi = (np.logical_and(x2 == 0, x1 < 0).astype(np.float32)
             * np.float32(PI)).astype(np.float32)
    phi = (phi + is_pi).astype(np.float32)
    phi2 = (np.round(r).astype(np.float32) * np.float32(2.0)
            * np.float32(PI) + phi).astype(np.float32)
    idx = np.argsort(phi2.reshape(-1), kind="stable")[:L]
    return size, idx.astype(np.int64)


_SIZE, _IDX = _spiral_pattern(_L)
_S2 = _SIZE * _SIZE

_NW = 32
_CHUNK = 128

_rows = (np.arange(_B, dtype=np.int64)[:, None] * _S2 + _IDX[None, :]).reshape(-1)
_N_SCHUNK = (_B * _L) // (_NW * _CHUNK)
_SIDX_NP = _rows.reshape(_NW, _N_SCHUNK, _CHUNK).astype(np.int32)

_mask = np.ones(_S2, dtype=bool)
_mask[_IDX] = False
_comp = np.nonzero(_mask)[0].astype(np.int64)
_HEAD = int(_IDX.min())
_TAIL0 = int(_IDX.max()) + 1
_TAILN = _S2 - _TAIL0
assert _HEAD == 646 and _TAILN == 734
_zrows = (np.arange(_B, dtype=np.int64)[:, None]) * _S2 + _comp[None, :]
_zrows = _zrows.reshape(-1)
_N_ZCHUNK = -(-len(_zrows) // (_NW * _CHUNK))
_pad = _N_ZCHUNK * _NW * _CHUNK - len(_zrows)
_zrows = np.concatenate([_zrows, _zrows[:_pad]])
_ZIDX_NP = _zrows.reshape(_NW, _N_ZCHUNK, _CHUNK).astype(np.int32)

_ROWS_PER_W = _N_SCHUNK * _CHUNK
_NBUF = 6
_GROUPS = [list(range(0, 6)), list(range(6, 12)), list(range(12, 16))]


def _make_scatter():
    mesh = plsc.VectorSubcoreMesh(core_axis_name="c", subcore_axis_name="s")

    @functools.partial(
        pl.kernel,
        mesh=mesh,
        out_type=jax.ShapeDtypeStruct((_B * _S2, _C), jnp.float32),
        scratch_types=[
            pltpu.VMEM((_N_SCHUNK, _CHUNK), jnp.int32),
            pltpu.VMEM((_N_ZCHUNK, _CHUNK), jnp.int32),
            pltpu.VMEM((_CHUNK, _C), jnp.float32),
        ] + [pltpu.VMEM((_CHUNK, _C), jnp.float32)] * _NBUF + [
            pltpu.SemaphoreType.DMA,
            pltpu.SemaphoreType.DMA,
            pltpu.SemaphoreType.DMA,
        ],
    )
    def scatter(in_hbm, sidx_hbm, zidx_hbm, zeros_hbm, out_hbm,
                sidx_v, zidx_v, zbuf_v, *rest):
        bufs = list(rest[:_NBUF])
        sem_in, sem_out, sem_z = rest[_NBUF:_NBUF + 3]

        nc = 2
        wid = lax.axis_index("s") * nc + lax.axis_index("c")
        base = wid * _ROWS_PER_W

        m0 = pltpu.async_copy(sidx_hbm.at[wid], sidx_v, sem_z)
        m1 = pltpu.async_copy(zidx_hbm.at[wid], zidx_v, sem_z)
        m2 = pltpu.async_copy(zeros_hbm, zbuf_v, sem_z)
        m0.wait(); m1.wait(); m2.wait()

        for gi, grp in enumerate(_GROUPS):
            st = [pltpu.async_copy(
                      in_hbm.at[pl.ds(base + c * _CHUNK, _CHUNK)],
                      bufs[j], sem_in)
                  for j, c in enumerate(grp)]
            for d in st:
                d.wait()
            plsc.subcore_barrier()
            sc = [pltpu.async_copy(
                      bufs[j], out_hbm.at[sidx_v.at[c]], sem_out)
                  for j, c in enumerate(grp)]
            if gi == len(_GROUPS) - 1:
                sc += [pltpu.async_copy(
                           zbuf_v, out_hbm.at[zidx_v.at[z]], sem_out)
                       for z in range(_N_ZCHUNK)]
            for d in sc:
                d.wait()
            plsc.subcore_barrier()

    return scatter


_scatter = _make_scatter()


def kernel(inputs):
    B, L, C = inputs.shape
    flat = inputs.reshape(B * L, C)
    sidx = jnp.asarray(_SIDX_NP)
    zidx = jnp.asarray(_ZIDX_NP)
    zeros = jnp.zeros((_CHUNK, _C), dtype=jnp.float32)
    out = _scatter(flat, sidx, zidx, zeros)
    return out.reshape(B, _SIZE, _SIZE, C)


# speedup vs baseline: 1.0229x; 1.0011x over previous
import functools

import jax
import jax.numpy as jnp
import numpy as np
from jax import lax
from jax.experimental import pallas as pl
from jax.experimental.pallas import tpu as pltpu
from jax.experimental.pallas import tpu_sc as plsc

_B, _L, _C = 16, 4096, 128


def _spiral_pattern(L):
    PI = float(np.arccos(0.0) * 2.0)
    size = np.sqrt(L / (PI / 4.0 * 0.7))
    size = np.round(size / 2.0)
    size = int(size * 2 + 1)
    rnge = (np.arange(size, dtype=np.float32) - np.float32(size / 2.0)
            + np.float32(0.5)).astype(np.float32)
    x1, x2 = np.meshgrid(rnge, rnge)
    r = np.sqrt(np.abs(x1 * x1 + x2 * x2), dtype=np.float32)
    with np.errstate(invalid="ignore", divide="ignore"):
        phi = np.arccos((x1 / r).astype(np.float32)).astype(np.float32)
    phi = np.where(np.isnan(phi), np.float32(0.0), phi)
    phi = (phi * np.sign(x2)).astype(np.float32)
    is_pi = (np.logical_and(x2 == 0, x1 < 0).astype(np.float32)
             * np.float32(PI)).astype(np.float32)
    phi = (phi + is_pi).astype(np.float32)
    phi2 = (np.round(r).astype(np.float32) * np.float32(2.0)
            * np.float32(PI) + phi).astype(np.float32)
    idx = np.argsort(phi2.reshape(-1), kind="stable")[:L]
    return size, idx.astype(np.int64)


_SIZE, _IDX = _spiral_pattern(_L)
_S2 = _SIZE * _SIZE

_NW = 32
_CHUNK = 128

_rows = (np.arange(_B, dtype=np.int64)[:, None] * _S2 + _IDX[None, :]).reshape(-1)
_N_SCHUNK = (_B * _L) // (_NW * _CHUNK)
_SIDX_NP = _rows.reshape(_NW, _N_SCHUNK, _CHUNK).astype(np.int32)

_mask = np.ones(_S2, dtype=bool)
_mask[_IDX] = False
_comp = np.nonzero(_mask)[0].astype(np.int64)
_zrows = (np.arange(_B, dtype=np.int64)[:, None]) * _S2 + _comp[None, :]
_zrows = _zrows.reshape(-1)
_N_ZCHUNK = -(-len(_zrows) // (_NW * _CHUNK))
_pad = _N_ZCHUNK * _NW * _CHUNK - len(_zrows)
_zrows = np.concatenate([_zrows, _zrows[:_pad]])
_ZIDX_NP = _zrows.reshape(_NW, _N_ZCHUNK, _CHUNK).astype(np.int32)

_ROWS_PER_W = _N_SCHUNK * _CHUNK
_WIN = 6 * _CHUNK
_ZOFF = _WIN
_PHASES = [list(range(0, 6)), list(range(6, 12)), list(range(12, 16))]


def _make_scatter():
    mesh = plsc.VectorSubcoreMesh(core_axis_name="c", subcore_axis_name="s")

    @functools.partial(
        pl.kernel,
        mesh=mesh,
        out_type=jax.ShapeDtypeStruct((_B * _S2, _C), jnp.float32),
        scratch_types=[
            pltpu.VMEM((_N_SCHUNK, _CHUNK), jnp.int32),
            pltpu.VMEM((_N_ZCHUNK, _CHUNK), jnp.int32),
            pltpu.VMEM((_WIN + _CHUNK, _C), jnp.float32),
            pltpu.SemaphoreType.DMA,
            pltpu.SemaphoreType.DMA,
        ],
    )
    def scatter(in_hbm, sidx_hbm, zidx_hbm, zeros_hbm, out_hbm,
                sidx_v, zidx_v, buf_v, sem_in, sem_out):
        nc = 2
        wid = lax.axis_index("s") * nc + lax.axis_index("c")
        base = wid * _ROWS_PER_W

        m0 = pltpu.async_copy(sidx_hbm.at[wid], sidx_v, sem_in)
        m1 = pltpu.async_copy(zidx_hbm.at[wid], zidx_v, sem_in)
        m2 = pltpu.async_copy(zeros_hbm, buf_v.at[pl.ds(_ZOFF, _CHUNK)],
                              sem_in)
        m0.wait(); m1.wait(); m2.wait()

        for pi, chunks in enumerate(_PHASES):
            c0, n = chunks[0], len(chunks)
            pltpu.async_copy(
                in_hbm.at[pl.ds(base + c0 * _CHUNK, n * _CHUNK)],
                buf_v.at[pl.ds(0, n * _CHUNK)], sem_in).wait()
            plsc.subcore_barrier()
            sc = [pltpu.async_copy(
                      buf_v.at[pl.ds(j * _CHUNK, _CHUNK)],
                      out_hbm.at[sidx_v.at[c]], sem_out)
                  for j, c in enumerate(chunks)]
            if pi == 0:
                sc += [pltpu.async_copy(
                           buf_v.at[pl.ds(_ZOFF, _CHUNK)],
                           out_hbm.at[zidx_v.at[z]], sem_out)
                       for z in range(_N_ZCHUNK)]
            for d in sc:
                d.wait()
            plsc.subcore_barrier()

    return scatter


_scatter = _make_scatter()


def kernel(inputs):
    B, L, C = inputs.shape
    flat = inputs.reshape(B * L, C)
    sidx = jnp.asarray(_SIDX_NP)
    zidx = jnp.asarray(_ZIDX_NP)
    zeros = jnp.zeros((_CHUNK, _C), dtype=jnp.float32)
    out = _scatter(flat, sidx, zidx, zeros)
    return out.reshape(B, _SIZE, _SIZE, C)
